# baseline (speedup 1.0000x reference)
import functools

import jax
import jax.numpy as jnp
from jax.experimental import pallas as pl
from jax.experimental.pallas import tpu as pltpu

P = 4


def _lstm_body(x_ref, nrow_ref, cnt_ref, w_ref, wc_ref, out_ref, h_s, c_s):
    G, T, RPB, DX = x_ref.shape
    DH4 = wc_ref.shape[1]
    R = G * RPB
    Wfull = w_ref[...]
    wco = wc_ref[0:1, :]
    ones = jnp.ones((R, 1), jnp.bfloat16)

    def step(t, carry):
        H, C = carry
        xt = x_ref[:, t, :, :].reshape(R, DX).astype(jnp.bfloat16)
        A = jnp.concatenate([H, C.astype(jnp.bfloat16), xt, ones], axis=1)
        g = jnp.dot(A, Wfull, preferred_element_type=jnp.float32)
        I = 0.5 * jnp.tanh(g[:, 0:DH4]) + 0.5
        F = 0.5 * jnp.tanh(g[:, DH4:2 * DH4]) + 0.5
        Tg = jnp.tanh(g[:, 2 * DH4:3 * DH4])
        C2 = F * C + I * Tg
        O = 0.5 * jnp.tanh(g[:, 3 * DH4:4 * DH4] + wco * C2) + 0.5
        H2 = (O * jnp.tanh(C2)).astype(jnp.bfloat16)
        return H2, C2

    tc = pl.program_id(1)
    ntc = pl.num_programs(1)

    @pl.when(tc == 0)
    def _init():
        h_s[...] = jnp.zeros((R, DH4), jnp.bfloat16)
        c_s[...] = jnp.zeros((R, DH4), jnp.float32)

    H, C = jax.lax.fori_loop(0, T, step, (h_s[...], c_s[...]), unroll=2)
    h_s[...] = H
    c_s[...] = C

    @pl.when(tc == ntc - 1)
    def _pool():
        D_H = DH4 // P
        r_iota = jax.lax.broadcasted_iota(jnp.int32, (R, DH4), 0)
        l_iota = jax.lax.broadcasted_iota(jnp.int32, (R, DH4), 1)
        node = (r_iota % RPB) * P + l_iota // D_H
        Hm = H.astype(jnp.float32) * (node < nrow_ref[...]).astype(jnp.float32)
        s = Hm.reshape(G, RPB, DH4).sum(axis=1)
        s = (s[:, 0:D_H] + s[:, D_H:2 * D_H]
             + s[:, 2 * D_H:3 * D_H] + s[:, 3 * D_H:4 * D_H])
        out_ref[...] = s / cnt_ref[...]


def kernel(agent_obs, hideout_obs, timestep_obs, num_agents,
           W_i, Wh_i, bh_i, b_i, w_c_i,
           W_f, Wh_f, bh_f, b_f, w_c_f,
           W_c, Wh_c, bh_c, b_c,
           W_o, Wh_o, bh_o, b_o, w_c_o):
    B, T, MAX_N, D_IN = agent_obs.shape
    D_H = W_i.shape[1]
    RPB = MAX_N // P
    G = 64
    TCH = 10

    x = agent_obs.reshape(B, T, RPB, P * D_IN)
    eye = jnp.eye(P, dtype=jnp.float32)
    gate_s = (0.5, 0.5, 1.0, 0.5)
    Wbd = jnp.concatenate(
        [s * jnp.kron(eye, Wg)
         for s, Wg in zip(gate_s, (W_i, W_f, W_c, W_o))], axis=1)
    Whbd = jnp.concatenate(
        [s * jnp.kron(eye, Wg)
         for s, Wg in zip(gate_s, (Wh_i, Wh_f, Wh_c, Wh_o))], axis=1)
    bias = jnp.concatenate(
        [s * jnp.tile(bh + b.reshape(-1), P)
         for s, (bh, b) in zip(gate_s, ((bh_i, b_i), (bh_f, b_f),
                                        (bh_c, b_c), (bh_o, b_o)))]
    ).reshape(1, 4 * P * D_H)
    peep_if = jnp.concatenate(
        [jnp.diag(0.5 * jnp.tile(w_c_i.reshape(-1), P)),
         jnp.diag(0.5 * jnp.tile(w_c_f.reshape(-1), P)),
         jnp.zeros((P * D_H, 2 * P * D_H), jnp.float32)], axis=1)
    Wfull = jnp.concatenate([Whbd, peep_if, Wbd, bias],
                            axis=0).astype(jnp.bfloat16)
    wco = (0.5 * jnp.tile(w_c_o.reshape(-1), P)).reshape(1, P * D_H)
    na_i32 = num_agents.astype(jnp.int32)
    na_row = jnp.repeat(na_i32, RPB).reshape(B * RPB, 1)
    counts = na_i32.astype(jnp.float32).reshape(B, 1)

    pooled = pl.pallas_call(
        _lstm_body,
        grid=(B // G, T // TCH),
        in_specs=[
            pl.BlockSpec((G, TCH, RPB, P * D_IN), lambda i, tc: (i, tc, 0, 0)),
            pl.BlockSpec((G * RPB, 1), lambda i, tc: (i, 0)),
            pl.BlockSpec((G, 1), lambda i, tc: (i, 0)),
            pl.BlockSpec(Wfull.shape, lambda i, tc: (0, 0)),
            pl.BlockSpec((1, P * D_H), lambda i, tc: (0, 0)),
        ],
        out_specs=pl.BlockSpec((G, D_H), lambda i, tc: (i, 0)),
        out_shape=jax.ShapeDtypeStruct((B, D_H), jnp.float32),
        scratch_shapes=[
            pltpu.VMEM((G * RPB, P * D_H), jnp.bfloat16),
            pltpu.VMEM((G * RPB, P * D_H), jnp.float32),
        ],
    )(x, na_row, counts, Wfull, wco)

    return jnp.concatenate([pooled, hideout_obs, timestep_obs], axis=-1)


# speedup vs baseline: 1.0912x; 1.0912x over previous
import functools

import jax
import jax.numpy as jnp
from jax.experimental import pallas as pl
from jax.experimental.pallas import tpu as pltpu

P = 4


def _lstm_body(x_ref, nrow_ref, cnt_ref, w_ref, wc_ref, out_ref, h_s, c_s):
    G, T, RPB, DX = x_ref.shape
    DH4 = wc_ref.shape[1]
    R = G * RPB
    Wfull = w_ref[...]
    wco = wc_ref[0:1, :]
    ones = jnp.ones((R, 1), jnp.bfloat16)

    def step(t, carry):
        H, C = carry
        xt = x_ref[:, t, :, :].reshape(R, DX).astype(jnp.bfloat16)
        A = jnp.concatenate([H, C.astype(jnp.bfloat16), xt, ones], axis=1)
        g = jnp.dot(A, Wfull, preferred_element_type=jnp.float32)
        I = 0.5 * jnp.tanh(g[:, 0:DH4]) + 0.5
        F = 0.5 * jnp.tanh(g[:, DH4:2 * DH4]) + 0.5
        Tg = jnp.tanh(g[:, 2 * DH4:3 * DH4])
        C2 = F * C + I * Tg
        O = 0.5 * jnp.tanh(g[:, 3 * DH4:4 * DH4] + wco * C2) + 0.5
        H2 = (O * jnp.tanh(C2)).astype(jnp.bfloat16)
        return H2, C2

    tc = pl.program_id(1)
    ntc = pl.num_programs(1)

    @pl.when(tc == 0)
    def _init():
        h_s[...] = jnp.zeros((R, DH4), jnp.bfloat16)
        c_s[...] = jnp.zeros((R, DH4), jnp.float32)

    H, C = jax.lax.fori_loop(0, T, step, (h_s[...], c_s[...]), unroll=5)
    h_s[...] = H
    c_s[...] = C

    @pl.when(tc == ntc - 1)
    def _pool():
        D_H = DH4 // P
        r_iota = jax.lax.broadcasted_iota(jnp.int32, (R, DH4), 0)
        l_iota = jax.lax.broadcasted_iota(jnp.int32, (R, DH4), 1)
        node = (r_iota % RPB) * P + l_iota // D_H
        Hm = H.astype(jnp.float32) * (node < nrow_ref[...]).astype(jnp.float32)
        s = Hm.reshape(G, RPB, DH4).sum(axis=1)
        s = (s[:, 0:D_H] + s[:, D_H:2 * D_H]
             + s[:, 2 * D_H:3 * D_H] + s[:, 3 * D_H:4 * D_H])
        out_ref[...] = s / cnt_ref[...]


def kernel(agent_obs, hideout_obs, timestep_obs, num_agents,
           W_i, Wh_i, bh_i, b_i, w_c_i,
           W_f, Wh_f, bh_f, b_f, w_c_f,
           W_c, Wh_c, bh_c, b_c,
           W_o, Wh_o, bh_o, b_o, w_c_o):
    B, T, MAX_N, D_IN = agent_obs.shape
    D_H = W_i.shape[1]
    RPB = MAX_N // P
    G = 32
    TCH = 50

    x = agent_obs.reshape(B, T, RPB, P * D_IN)
    eye = jnp.eye(P, dtype=jnp.float32)
    gate_s = (0.5, 0.5, 1.0, 0.5)
    Wbd = jnp.concatenate(
        [s * jnp.kron(eye, Wg)
         for s, Wg in zip(gate_s, (W_i, W_f, W_c, W_o))], axis=1)
    Whbd = jnp.concatenate(
        [s * jnp.kron(eye, Wg)
         for s, Wg in zip(gate_s, (Wh_i, Wh_f, Wh_c, Wh_o))], axis=1)
    bias = jnp.concatenate(
        [s * jnp.tile(bh + b.reshape(-1), P)
         for s, (bh, b) in zip(gate_s, ((bh_i, b_i), (bh_f, b_f),
                                        (bh_c, b_c), (bh_o, b_o)))]
    ).reshape(1, 4 * P * D_H)
    peep_if = jnp.concatenate(
        [jnp.diag(0.5 * jnp.tile(w_c_i.reshape(-1), P)),
         jnp.diag(0.5 * jnp.tile(w_c_f.reshape(-1), P)),
         jnp.zeros((P * D_H, 2 * P * D_H), jnp.float32)], axis=1)
    Wfull = jnp.concatenate([Whbd, peep_if, Wbd, bias],
                            axis=0).astype(jnp.bfloat16)
    wco = (0.5 * jnp.tile(w_c_o.reshape(-1), P)).reshape(1, P * D_H)
    na_i32 = num_agents.astype(jnp.int32)
    na_row = jnp.repeat(na_i32, RPB).reshape(B * RPB, 1)
    counts = na_i32.astype(jnp.float32).reshape(B, 1)

    pooled = pl.pallas_call(
        _lstm_body,
        grid=(B // G, T // TCH),
        in_specs=[
            pl.BlockSpec((G, TCH, RPB, P * D_IN), lambda i, tc: (i, tc, 0, 0)),
            pl.BlockSpec((G * RPB, 1), lambda i, tc: (i, 0)),
            pl.BlockSpec((G, 1), lambda i, tc: (i, 0)),
            pl.BlockSpec(Wfull.shape, lambda i, tc: (0, 0)),
            pl.BlockSpec((1, P * D_H), lambda i, tc: (0, 0)),
        ],
        out_specs=pl.BlockSpec((G, D_H), lambda i, tc: (i, 0)),
        out_shape=jax.ShapeDtypeStruct((B, D_H), jnp.float32),
        scratch_shapes=[
            pltpu.VMEM((G * RPB, P * D_H), jnp.bfloat16),
            pltpu.VMEM((G * RPB, P * D_H), jnp.float32),
        ],
    )(x, na_row, counts, Wfull, wco)

    return jnp.concatenate([pooled, hideout_obs, timestep_obs], axis=-1)


# speedup vs baseline: 1.1254x; 1.0314x over previous
import functools

import jax
import jax.numpy as jnp
from jax.experimental import pallas as pl
from jax.experimental.pallas import tpu as pltpu

P = 4


def _lstm_body(x_ref, nrow_ref, cnt_ref, w_ref, wc_ref, out_ref, h_s, c_s):
    G, T, RPB, DX = x_ref.shape
    DH4 = wc_ref.shape[1]
    R = G * RPB
    Wfull = w_ref[...]
    wco = wc_ref[0:1, :]
    ones = jnp.ones((R, 1), jnp.bfloat16)

    def step(t, carry):
        H, C = carry
        xt = x_ref[:, t, :, :].reshape(R, DX).astype(jnp.bfloat16)
        A = jnp.concatenate([H, C.astype(jnp.bfloat16), xt, ones], axis=1)
        g = jnp.dot(A, Wfull, preferred_element_type=jnp.float32)
        I = 0.5 * jnp.tanh(g[:, 0:DH4]) + 0.5
        F = 0.5 * jnp.tanh(g[:, DH4:2 * DH4]) + 0.5
        Tg = jnp.tanh(g[:, 2 * DH4:3 * DH4])
        C2 = F * C + I * Tg
        O = 0.5 * jnp.tanh(g[:, 3 * DH4:4 * DH4] + wco * C2) + 0.5
        H2 = (O * jnp.tanh(C2)).astype(jnp.bfloat16)
        return H2, C2

    tc = pl.program_id(1)
    ntc = pl.num_programs(1)

    @pl.when(tc == 0)
    def _init():
        h_s[...] = jnp.zeros((R, DH4), jnp.bfloat16)
        c_s[...] = jnp.zeros((R, DH4), jnp.float32)

    H, C = jax.lax.fori_loop(0, T, step, (h_s[...], c_s[...]), unroll=10)
    h_s[...] = H
    c_s[...] = C

    @pl.when(tc == ntc - 1)
    def _pool():
        D_H = DH4 // P
        r_iota = jax.lax.broadcasted_iota(jnp.int32, (R, DH4), 0)
        l_iota = jax.lax.broadcasted_iota(jnp.int32, (R, DH4), 1)
        node = (r_iota % RPB) * P + l_iota // D_H
        Hm = H.astype(jnp.float32) * (node < nrow_ref[...]).astype(jnp.float32)
        s = Hm.reshape(G, RPB, DH4).sum(axis=1)
        s = (s[:, 0:D_H] + s[:, D_H:2 * D_H]
             + s[:, 2 * D_H:3 * D_H] + s[:, 3 * D_H:4 * D_H])
        out_ref[...] = s / cnt_ref[...]


def kernel(agent_obs, hideout_obs, timestep_obs, num_agents,
           W_i, Wh_i, bh_i, b_i, w_c_i,
           W_f, Wh_f, bh_f, b_f, w_c_f,
           W_c, Wh_c, bh_c, b_c,
           W_o, Wh_o, bh_o, b_o, w_c_o):
    B, T, MAX_N, D_IN = agent_obs.shape
    D_H = W_i.shape[1]
    RPB = MAX_N // P
    G = 32
    TCH = 50

    x = agent_obs.reshape(B, T, RPB, P * D_IN)
    eye = jnp.eye(P, dtype=jnp.float32)
    gate_s = (0.5, 0.5, 1.0, 0.5)
    Wbd = jnp.concatenate(
        [s * jnp.kron(eye, Wg)
         for s, Wg in zip(gate_s, (W_i, W_f, W_c, W_o))], axis=1)
    Whbd = jnp.concatenate(
        [s * jnp.kron(eye, Wg)
         for s, Wg in zip(gate_s, (Wh_i, Wh_f, Wh_c, Wh_o))], axis=1)
    bias = jnp.concatenate(
        [s * jnp.tile(bh + b.reshape(-1), P)
         for s, (bh, b) in zip(gate_s, ((bh_i, b_i), (bh_f, b_f),
                                        (bh_c, b_c), (bh_o, b_o)))]
    ).reshape(1, 4 * P * D_H)
    peep_if = jnp.concatenate(
        [jnp.diag(0.5 * jnp.tile(w_c_i.reshape(-1), P)),
         jnp.diag(0.5 * jnp.tile(w_c_f.reshape(-1), P)),
         jnp.zeros((P * D_H, 2 * P * D_H), jnp.float32)], axis=1)
    Wfull = jnp.concatenate([Whbd, peep_if, Wbd, bias],
                            axis=0).astype(jnp.bfloat16)
    wco = (0.5 * jnp.tile(w_c_o.reshape(-1), P)).reshape(1, P * D_H)
    na_i32 = num_agents.astype(jnp.int32)
    na_row = jnp.repeat(na_i32, RPB).reshape(B * RPB, 1)
    counts = na_i32.astype(jnp.float32).reshape(B, 1)

    pooled = pl.pallas_call(
        _lstm_body,
        grid=(B // G, T // TCH),
        in_specs=[
            pl.BlockSpec((G, TCH, RPB, P * D_IN), lambda i, tc: (i, tc, 0, 0)),
            pl.BlockSpec((G * RPB, 1), lambda i, tc: (i, 0)),
            pl.BlockSpec((G, 1), lambda i, tc: (i, 0)),
            pl.BlockSpec(Wfull.shape, lambda i, tc: (0, 0)),
            pl.BlockSpec((1, P * D_H), lambda i, tc: (0, 0)),
        ],
        out_specs=pl.BlockSpec((G, D_H), lambda i, tc: (i, 0)),
        out_shape=jax.ShapeDtypeStruct((B, D_H), jnp.float32),
        scratch_shapes=[
            pltpu.VMEM((G * RPB, P * D_H), jnp.bfloat16),
            pltpu.VMEM((G * RPB, P * D_H), jnp.float32),
        ],
    )(x, na_row, counts, Wfull, wco)

    return jnp.concatenate([pooled, hideout_obs, timestep_obs], axis=-1)


# speedup vs baseline: 1.1437x; 1.0163x over previous
import functools

import jax
import jax.numpy as jnp
from jax.experimental import pallas as pl
from jax.experimental.pallas import tpu as pltpu

P = 4


def _lstm_body(x_ref, nrow_ref, cnt_ref, w_ref, wc_ref, out_ref, h_s, c_s):
    G, T, RPB, DX = x_ref.shape
    DH4 = wc_ref.shape[1]
    R = G * RPB
    Wfull = w_ref[...]
    wco = wc_ref[0:1, :]
    ones = jnp.ones((R, 1), jnp.bfloat16)

    def step(t, carry):
        H, C = carry
        xt = x_ref[:, t, :, :].reshape(R, DX).astype(jnp.bfloat16)
        A = jnp.concatenate([H, C.astype(jnp.bfloat16), xt, ones], axis=1)
        g = jnp.dot(A, Wfull, preferred_element_type=jnp.float32)
        I = 0.5 * jnp.tanh(g[:, 0:DH4]) + 0.5
        F = 0.5 * jnp.tanh(g[:, DH4:2 * DH4]) + 0.5
        Tg = jnp.tanh(g[:, 2 * DH4:3 * DH4])
        C2 = F * C + I * Tg
        O = 0.5 * jnp.tanh(g[:, 3 * DH4:4 * DH4] + wco * C2) + 0.5
        H2 = (O * jnp.tanh(C2)).astype(jnp.bfloat16)
        return H2, C2

    tc = pl.program_id(1)
    ntc = pl.num_programs(1)

    @pl.when(tc == 0)
    def _init():
        h_s[...] = jnp.zeros((R, DH4), jnp.bfloat16)
        c_s[...] = jnp.zeros((R, DH4), jnp.float32)

    H, C = jax.lax.fori_loop(0, T, step, (h_s[...], c_s[...]), unroll=25)
    h_s[...] = H
    c_s[...] = C

    @pl.when(tc == ntc - 1)
    def _pool():
        D_H = DH4 // P
        r_iota = jax.lax.broadcasted_iota(jnp.int32, (R, DH4), 0)
        l_iota = jax.lax.broadcasted_iota(jnp.int32, (R, DH4), 1)
        node = (r_iota % RPB) * P + l_iota // D_H
        Hm = H.astype(jnp.float32) * (node < nrow_ref[...]).astype(jnp.float32)
        s = Hm.reshape(G, RPB, DH4).sum(axis=1)
        s = (s[:, 0:D_H] + s[:, D_H:2 * D_H]
             + s[:, 2 * D_H:3 * D_H] + s[:, 3 * D_H:4 * D_H])
        out_ref[...] = s / cnt_ref[...]


def kernel(agent_obs, hideout_obs, timestep_obs, num_agents,
           W_i, Wh_i, bh_i, b_i, w_c_i,
           W_f, Wh_f, bh_f, b_f, w_c_f,
           W_c, Wh_c, bh_c, b_c,
           W_o, Wh_o, bh_o, b_o, w_c_o):
    B, T, MAX_N, D_IN = agent_obs.shape
    D_H = W_i.shape[1]
    RPB = MAX_N // P
    G = 32
    TCH = 50

    x = agent_obs.reshape(B, T, RPB, P * D_IN)
    eye = jnp.eye(P, dtype=jnp.float32)
    gate_s = (0.5, 0.5, 1.0, 0.5)
    Wbd = jnp.concatenate(
        [s * jnp.kron(eye, Wg)
         for s, Wg in zip(gate_s, (W_i, W_f, W_c, W_o))], axis=1)
    Whbd = jnp.concatenate(
        [s * jnp.kron(eye, Wg)
         for s, Wg in zip(gate_s, (Wh_i, Wh_f, Wh_c, Wh_o))], axis=1)
    bias = jnp.concatenate(
        [s * jnp.tile(bh + b.reshape(-1), P)
         for s, (bh, b) in zip(gate_s, ((bh_i, b_i), (bh_f, b_f),
                                        (bh_c, b_c), (bh_o, b_o)))]
    ).reshape(1, 4 * P * D_H)
    peep_if = jnp.concatenate(
        [jnp.diag(0.5 * jnp.tile(w_c_i.reshape(-1), P)),
         jnp.diag(0.5 * jnp.tile(w_c_f.reshape(-1), P)),
         jnp.zeros((P * D_H, 2 * P * D_H), jnp.float32)], axis=1)
    Wfull = jnp.concatenate([Whbd, peep_if, Wbd, bias],
                            axis=0).astype(jnp.bfloat16)
    wco = (0.5 * jnp.tile(w_c_o.reshape(-1), P)).reshape(1, P * D_H)
    na_i32 = num_agents.astype(jnp.int32)
    na_row = jnp.repeat(na_i32, RPB).reshape(B * RPB, 1)
    counts = na_i32.astype(jnp.float32).reshape(B, 1)

    pooled = pl.pallas_call(
        _lstm_body,
        grid=(B // G, T // TCH),
        in_specs=[
            pl.BlockSpec((G, TCH, RPB, P * D_IN), lambda i, tc: (i, tc, 0, 0)),
            pl.BlockSpec((G * RPB, 1), lambda i, tc: (i, 0)),
            pl.BlockSpec((G, 1), lambda i, tc: (i, 0)),
            pl.BlockSpec(Wfull.shape, lambda i, tc: (0, 0)),
            pl.BlockSpec((1, P * D_H), lambda i, tc: (0, 0)),
        ],
        out_specs=pl.BlockSpec((G, D_H), lambda i, tc: (i, 0)),
        out_shape=jax.ShapeDtypeStruct((B, D_H), jnp.float32),
        scratch_shapes=[
            pltpu.VMEM((G * RPB, P * D_H), jnp.bfloat16),
            pltpu.VMEM((G * RPB, P * D_H), jnp.float32),
        ],
    )(x, na_row, counts, Wfull, wco)

    return jnp.concatenate([pooled, hideout_obs, timestep_obs], axis=-1)


# speedup vs baseline: 1.1444x; 1.0007x over previous
import jax
import jax.numpy as jnp
from jax.experimental import pallas as pl
from jax.experimental.pallas import tpu as pltpu

P = 4


def _lstm_body(x_ref, nrow_ref, cnt_ref, w_ref, wc_ref, out_ref, h_s, c_s):
    G, T, RPB, DX = x_ref.shape
    DH4 = wc_ref.shape[1]
    R = G * RPB
    Wfull = w_ref[...]
    wco = wc_ref[0:1, :]
    ones = jnp.ones((R, 1), jnp.bfloat16)

    def step(t, carry):
        H, C = carry
        xt = x_ref[:, t, :, :].reshape(R, DX).astype(jnp.bfloat16)
        A = jnp.concatenate([H, C.astype(jnp.bfloat16), xt, ones], axis=1)
        g = jnp.dot(A, Wfull, preferred_element_type=jnp.float32)
        I = 0.5 * jnp.tanh(g[:, 0:DH4]) + 0.5
        F = 0.5 * jnp.tanh(g[:, DH4:2 * DH4]) + 0.5
        Tg = jnp.tanh(g[:, 2 * DH4:3 * DH4])
        C2 = F * C + I * Tg
        O = 0.5 * jnp.tanh(g[:, 3 * DH4:4 * DH4] + wco * C2) + 0.5
        H2 = (O * jnp.tanh(C2)).astype(jnp.bfloat16)
        return H2, C2

    tc = pl.program_id(1)
    ntc = pl.num_programs(1)

    @pl.when(tc == 0)
    def _init():
        h_s[...] = jnp.zeros((R, DH4), jnp.bfloat16)
        c_s[...] = jnp.zeros((R, DH4), jnp.float32)

    H, C = jax.lax.fori_loop(0, T, step, (h_s[...], c_s[...]), unroll=25)
    h_s[...] = H
    c_s[...] = C

    @pl.when(tc == ntc - 1)
    def _pool():
        D_H = DH4 // P
        r_iota = jax.lax.broadcasted_iota(jnp.int32, (R, DH4), 0)
        l_iota = jax.lax.broadcasted_iota(jnp.int32, (R, DH4), 1)
        node = (r_iota % RPB) * P + l_iota // D_H
        Hm = H.astype(jnp.float32) * (node < nrow_ref[...]).astype(jnp.float32)
        s = Hm.reshape(G, RPB, DH4).sum(axis=1)
        s = (s[:, 0:D_H] + s[:, D_H:2 * D_H]
             + s[:, 2 * D_H:3 * D_H] + s[:, 3 * D_H:4 * D_H])
        out_ref[...] = s / cnt_ref[...]


def kernel(agent_obs, hideout_obs, timestep_obs, num_agents,
           W_i, Wh_i, bh_i, b_i, w_c_i,
           W_f, Wh_f, bh_f, b_f, w_c_f,
           W_c, Wh_c, bh_c, b_c,
           W_o, Wh_o, bh_o, b_o, w_c_o):
    B, T, MAX_N, D_IN = agent_obs.shape
    D_H = W_i.shape[1]
    RPB = MAX_N // P
    G = 32
    TCH = 50

    x = agent_obs.reshape(B, T, RPB, P * D_IN)
    eye = jnp.eye(P, dtype=jnp.float32)
    gate_s = (0.5, 0.5, 1.0, 0.5)
    Wbd = jnp.concatenate(
        [s * jnp.kron(eye, Wg)
         for s, Wg in zip(gate_s, (W_i, W_f, W_c, W_o))], axis=1)
    Whbd = jnp.concatenate(
        [s * jnp.kron(eye, Wg)
         for s, Wg in zip(gate_s, (Wh_i, Wh_f, Wh_c, Wh_o))], axis=1)
    bias = jnp.concatenate(
        [s * jnp.tile(bh + b.reshape(-1), P)
         for s, (bh, b) in zip(gate_s, ((bh_i, b_i), (bh_f, b_f),
                                        (bh_c, b_c), (bh_o, b_o)))]
    ).reshape(1, 4 * P * D_H)
    peep_if = jnp.concatenate(
        [jnp.diag(0.5 * jnp.tile(w_c_i.reshape(-1), P)),
         jnp.diag(0.5 * jnp.tile(w_c_f.reshape(-1), P)),
         jnp.zeros((P * D_H, 2 * P * D_H), jnp.float32)], axis=1)
    Wfull = jnp.concatenate([Whbd, peep_if, Wbd, bias],
                            axis=0).astype(jnp.bfloat16)
    wco = (0.5 * jnp.tile(w_c_o.reshape(-1), P)).reshape(1, P * D_H)
    na_i32 = num_agents.astype(jnp.int32)
    na_row = jnp.repeat(na_i32, RPB).reshape(B * RPB, 1)
    counts = na_i32.astype(jnp.float32).reshape(B, 1)

    pooled = pl.pallas_call(
        _lstm_body,
        grid=(B // G, T // TCH),
        in_specs=[
            pl.BlockSpec((G, TCH, RPB, P * D_IN), lambda i, tc: (i, tc, 0, 0)),
            pl.BlockSpec((G * RPB, 1), lambda i, tc: (i, 0)),
            pl.BlockSpec((G, 1), lambda i, tc: (i, 0)),
            pl.BlockSpec(Wfull.shape, lambda i, tc: (0, 0)),
            pl.BlockSpec((1, P * D_H), lambda i, tc: (0, 0)),
        ],
        out_specs=pl.BlockSpec((G, D_H), lambda i, tc: (i, 0)),
        out_shape=jax.ShapeDtypeStruct((B, D_H), jnp.float32),
        scratch_shapes=[
            pltpu.VMEM((G * RPB, P * D_H), jnp.bfloat16),
            pltpu.VMEM((G * RPB, P * D_H), jnp.float32),
        ],
    )(x, na_row, counts, Wfull, wco)

    return jnp.concatenate([pooled, hideout_obs, timestep_obs], axis=-1)


# speedup vs baseline: 1.1589x; 1.0126x over previous
import jax
import jax.numpy as jnp
from jax.experimental import pallas as pl
from jax.experimental.pallas import tpu as pltpu

P = 4


def _lstm_body(x_ref, nrow_ref, cnt_ref, w_ref, wc_ref, out_ref, h_s, c_s):
    G, T, RPB, DX = x_ref.shape
    DH4 = wc_ref.shape[1]
    R = G * RPB
    Wfull = w_ref[...]
    wco = wc_ref[0:1, :]
    ones = jnp.ones((R, 1), jnp.bfloat16)

    def step(t, carry):
        H, C = carry
        xt = x_ref[:, t, :, :].reshape(R, DX).astype(jnp.bfloat16)
        A = jnp.concatenate([H, C.astype(jnp.bfloat16), xt, ones], axis=1)
        g = jnp.dot(A, Wfull, preferred_element_type=jnp.float32)
        I = 0.5 * jnp.tanh(g[:, 0:DH4]) + 0.5
        F = 0.5 * jnp.tanh(g[:, DH4:2 * DH4]) + 0.5
        Tg = jnp.tanh(g[:, 2 * DH4:3 * DH4])
        C2 = F * C + I * Tg
        O = 0.5 * jnp.tanh(g[:, 3 * DH4:4 * DH4] + wco * C2) + 0.5
        H2 = (O * jnp.tanh(C2)).astype(jnp.bfloat16)
        return H2, C2

    tc = pl.program_id(1)
    ntc = pl.num_programs(1)

    @pl.when(tc == 0)
    def _init():
        h_s[...] = jnp.zeros((R, DH4), jnp.bfloat16)
        c_s[...] = jnp.zeros((R, DH4), jnp.float32)

    H, C = jax.lax.fori_loop(0, T, step, (h_s[...], c_s[...]), unroll=50)
    h_s[...] = H
    c_s[...] = C

    @pl.when(tc == ntc - 1)
    def _pool():
        D_H = DH4 // P
        r_iota = jax.lax.broadcasted_iota(jnp.int32, (R, DH4), 0)
        l_iota = jax.lax.broadcasted_iota(jnp.int32, (R, DH4), 1)
        node = (r_iota % RPB) * P + l_iota // D_H
        Hm = H.astype(jnp.float32) * (node < nrow_ref[...]).astype(jnp.float32)
        s = Hm.reshape(G, RPB, DH4).sum(axis=1)
        s = (s[:, 0:D_H] + s[:, D_H:2 * D_H]
             + s[:, 2 * D_H:3 * D_H] + s[:, 3 * D_H:4 * D_H])
        out_ref[...] = s / cnt_ref[...]


def kernel(agent_obs, hideout_obs, timestep_obs, num_agents,
           W_i, Wh_i, bh_i, b_i, w_c_i,
           W_f, Wh_f, bh_f, b_f, w_c_f,
           W_c, Wh_c, bh_c, b_c,
           W_o, Wh_o, bh_o, b_o, w_c_o):
    B, T, MAX_N, D_IN = agent_obs.shape
    D_H = W_i.shape[1]
    RPB = MAX_N // P
    G = 32
    TCH = 50

    x = agent_obs.reshape(B, T, RPB, P * D_IN)
    eye = jnp.eye(P, dtype=jnp.float32)
    gate_s = (0.5, 0.5, 1.0, 0.5)
    Wbd = jnp.concatenate(
        [s * jnp.kron(eye, Wg)
         for s, Wg in zip(gate_s, (W_i, W_f, W_c, W_o))], axis=1)
    Whbd = jnp.concatenate(
        [s * jnp.kron(eye, Wg)
         for s, Wg in zip(gate_s, (Wh_i, Wh_f, Wh_c, Wh_o))], axis=1)
    bias = jnp.concatenate(
        [s * jnp.tile(bh + b.reshape(-1), P)
         for s, (bh, b) in zip(gate_s, ((bh_i, b_i), (bh_f, b_f),
                                        (bh_c, b_c), (bh_o, b_o)))]
    ).reshape(1, 4 * P * D_H)
    peep_if = jnp.concatenate(
        [jnp.diag(0.5 * jnp.tile(w_c_i.reshape(-1), P)),
         jnp.diag(0.5 * jnp.tile(w_c_f.reshape(-1), P)),
         jnp.zeros((P * D_H, 2 * P * D_H), jnp.float32)], axis=1)
    Wfull = jnp.concatenate([Whbd, peep_if, Wbd, bias],
                            axis=0).astype(jnp.bfloat16)
    wco = (0.5 * jnp.tile(w_c_o.reshape(-1), P)).reshape(1, P * D_H)
    na_i32 = num_agents.astype(jnp.int32)
    na_row = jnp.repeat(na_i32, RPB).reshape(B * RPB, 1)
    counts = na_i32.astype(jnp.float32).reshape(B, 1)

    pooled = pl.pallas_call(
        _lstm_body,
        grid=(B // G, T // TCH),
        in_specs=[
            pl.BlockSpec((G, TCH, RPB, P * D_IN), lambda i, tc: (i, tc, 0, 0)),
            pl.BlockSpec((G * RPB, 1), lambda i, tc: (i, 0)),
            pl.BlockSpec((G, 1), lambda i, tc: (i, 0)),
            pl.BlockSpec(Wfull.shape, lambda i, tc: (0, 0)),
            pl.BlockSpec((1, P * D_H), lambda i, tc: (0, 0)),
        ],
        out_specs=pl.BlockSpec((G, D_H), lambda i, tc: (i, 0)),
        out_shape=jax.ShapeDtypeStruct((B, D_H), jnp.float32),
        scratch_shapes=[
            pltpu.VMEM((G * RPB, P * D_H), jnp.bfloat16),
            pltpu.VMEM((G * RPB, P * D_H), jnp.float32),
        ],
    )(x, na_row, counts, Wfull, wco)

    return jnp.concatenate([pooled, hideout_obs, timestep_obs], axis=-1)
